# baseline (speedup 1.0000x reference)
import functools

import jax
import jax.numpy as jnp
from jax import lax
from jax.experimental import pallas as pl
from jax.experimental.pallas import tpu as pltpu
from jax.experimental.pallas import tpu_sc as plsc

BATCH = 4096
SEQ = 200
EMBED_DIM = 64
PADDED = 2 * EMBED_DIM

NUM_CORES = 2
NUM_SUBCORES = 16
NUM_WORKERS = NUM_CORES * NUM_SUBCORES

CHUNK = 128
N_STEPS = SEQ
NBUF = 2
DIST = 2
MBUF = 2
LANES = 16

_mesh = plsc.VectorSubcoreMesh(
    core_axis_name="c", subcore_axis_name="s",
    num_cores=NUM_CORES, num_subcores=NUM_SUBCORES,
)


@functools.partial(
    pl.kernel,
    out_type=jax.ShapeDtypeStruct((SEQ, EMBED_DIM, BATCH), jnp.float32),
    mesh=_mesh,
    compiler_params=pltpu.CompilerParams(needs_layout_passes=False,
                                         disable_bounds_checks=True),
    scratch_types=(
        [pltpu.VMEM((N_STEPS, CHUNK), jnp.int32),
         pltpu.VMEM((NBUF, CHUNK, PADDED), jnp.float32),
         pltpu.VMEM((MBUF, EMBED_DIM, CHUNK), jnp.float32)]
        + [pltpu.SemaphoreType.DMA] * NBUF
        + [pltpu.SemaphoreType.DMA] * MBUF
    ),
)
def _emb_lookup(xt_hbm, w_hbm, out_hbm, idx_v, gbuf, tbuf, *sems):
    semg = sems[:NBUF]
    sems_ = sems[NBUF:]
    wid = lax.axis_index("s") * NUM_CORES + lax.axis_index("c")
    col0 = wid * CHUNK
    pltpu.sync_copy(xt_hbm.at[:, pl.ds(col0, CHUNK)], idx_v)

    iota = lax.iota(jnp.int32, LANES)
    c_vecs = [iota + (cg * LANES) for cg in range(CHUNK // LANES)]

    def fire_gather(s, b):
        pltpu.async_copy(w_hbm.at[idx_v.at[s]], gbuf.at[b], semg[b])

    def wait_gather(b):
        pltpu.make_async_copy(w_hbm.at[idx_v.at[0]], gbuf.at[b],
                              semg[b]).wait()

    def transpose_select(k, tb):
        src = gbuf.at[k]

        @plsc.parallel_loop(0, CHUNK, LANES, unroll=2)
        def tbody(c0):
            cv = iota + c0
            for e in range(EMBED_DIM):
                ev = jnp.full((LANES,), e, jnp.int32)
                vals = plsc.load_gather(src, [cv, ev])
                tbuf[tb, e, pl.ds(c0, LANES)] = vals

    def fire_scatter(s, tb):
        pltpu.async_copy(tbuf.at[tb],
                         out_hbm.at[s, :, pl.ds(col0, CHUNK)],
                         sems_[tb])

    def wait_scatter(tb):
        pltpu.make_async_copy(tbuf.at[tb],
                              out_hbm.at[0, :, pl.ds(col0, CHUNK)],
                              sems_[tb]).wait()

    def step(s, k, warm, refill):
        tb = k % MBUF
        wait_gather(k)
        if warm:
            wait_scatter(tb)
        transpose_select(k, tb)
        fire_scatter(s, tb)
        if refill:
            fire_gather(s + DIST, (k + DIST) % NBUF)

    for s in range(DIST):
        fire_gather(s, s)
    for s in range(NBUF):
        step(s, s, warm=(s >= MBUF), refill=True)

    def body(g, carry):
        s0 = g * NBUF
        for k in range(NBUF):
            step(s0 + k, k, warm=True, refill=True)
        return carry

    lax.fori_loop(1, N_STEPS // NBUF - 1, body, 0)

    for k in range(NBUF):
        s = N_STEPS - NBUF + k
        step(s, k, warm=True, refill=(s + DIST < N_STEPS))
    for tb in range(MBUF):
        wait_scatter(tb)


def kernel(x, W):
    xt = jnp.transpose(x).astype(jnp.int32)
    w128 = jnp.pad(W, ((0, 0), (0, PADDED - EMBED_DIM)))
    p = _emb_lookup(xt, w128)
    return jnp.transpose(p, (2, 0, 1))


# speedup vs baseline: 1.0730x; 1.0730x over previous
import functools

import jax
import jax.numpy as jnp
from jax import lax
from jax.experimental import pallas as pl
from jax.experimental.pallas import tpu as pltpu
from jax.experimental.pallas import tpu_sc as plsc

BATCH = 4096
SEQ = 200
EMBED_DIM = 64
PADDED = 2 * EMBED_DIM

NUM_CORES = 2
NUM_SUBCORES = 16
NUM_WORKERS = NUM_CORES * NUM_SUBCORES

CHUNK = 128
N_STEPS = SEQ
NBUF = 2
DIST = 2
MBUF = 2
LANES = 16

_mesh = plsc.VectorSubcoreMesh(
    core_axis_name="c", subcore_axis_name="s",
    num_cores=NUM_CORES, num_subcores=NUM_SUBCORES,
)


@functools.partial(
    pl.kernel,
    out_type=jax.ShapeDtypeStruct((SEQ, EMBED_DIM, BATCH), jnp.float32),
    mesh=_mesh,
    compiler_params=pltpu.CompilerParams(needs_layout_passes=False,
                                         disable_bounds_checks=True),
    scratch_types=(
        [pltpu.VMEM((N_STEPS, CHUNK), jnp.int32),
         pltpu.VMEM((NBUF, CHUNK), jnp.int32),
         pltpu.VMEM((NBUF, CHUNK, PADDED), jnp.float32),
         pltpu.VMEM((MBUF, EMBED_DIM, CHUNK), jnp.float32)]
        + [pltpu.SemaphoreType.DMA] * NBUF
        + [pltpu.SemaphoreType.DMA] * MBUF
    ),
)
def _emb_lookup(xt_hbm, w_hbm, out_hbm, idx_v, pair_v, gbuf, tbuf, *sems):
    semg = sems[:NBUF]
    sems_ = sems[NBUF:]
    wid = lax.axis_index("s") * NUM_CORES + lax.axis_index("c")
    col0 = wid * CHUNK
    pltpu.sync_copy(xt_hbm.at[:, pl.ds(col0, CHUNK)], idx_v)

    iota = lax.iota(jnp.int32, LANES)
    c_vecs = [iota + (cg * LANES) for cg in range(CHUNK // LANES)]

    def fire_gather(s, b):
        for cg in range(CHUNK // LANES):
            v = idx_v[s, pl.ds(cg * LANES, LANES)]
            pair_v[b, pl.ds(cg * LANES, LANES)] = lax.shift_right_logical(v, 1)
        pltpu.async_copy(w_hbm.at[pair_v.at[b]], gbuf.at[b], semg[b])

    def wait_gather(b):
        pltpu.make_async_copy(w_hbm.at[pair_v.at[b]], gbuf.at[b],
                              semg[b]).wait()

    def transpose_select(s, k, tb):
        src = gbuf.at[k]

        @plsc.parallel_loop(0, CHUNK, LANES)
        def tbody(c0):
            cv = iota + c0
            off = (idx_v[s, pl.ds(c0, LANES)] & 1) * EMBED_DIM
            for e in range(EMBED_DIM):
                vals = plsc.load_gather(src, [cv, off + e])
                tbuf[tb, e, pl.ds(c0, LANES)] = vals

    def fire_scatter(s, tb):
        pltpu.async_copy(tbuf.at[tb],
                         out_hbm.at[s, :, pl.ds(col0, CHUNK)],
                         sems_[tb])

    def wait_scatter(tb):
        pltpu.make_async_copy(tbuf.at[tb],
                              out_hbm.at[0, :, pl.ds(col0, CHUNK)],
                              sems_[tb]).wait()

    def step(s, k, warm, refill):
        tb = k % MBUF
        wait_gather(k)
        if warm:
            wait_scatter(tb)
        transpose_select(s, k, tb)
        fire_scatter(s, tb)
        if refill:
            fire_gather(s + DIST, (k + DIST) % NBUF)

    for s in range(DIST):
        fire_gather(s, s)
    for s in range(NBUF):
        step(s, s, warm=(s >= MBUF), refill=True)

    def body(g, carry):
        s0 = g * NBUF
        for k in range(NBUF):
            step(s0 + k, k, warm=True, refill=True)
        return carry

    lax.fori_loop(1, N_STEPS // NBUF - 1, body, 0)

    for k in range(NBUF):
        s = N_STEPS - NBUF + k
        step(s, k, warm=True, refill=(s + DIST < N_STEPS))
    for tb in range(MBUF):
        wait_scatter(tb)


def kernel(x, W):
    xt = jnp.transpose(x).astype(jnp.int32)
    w2 = W.reshape(500000, PADDED)
    p = _emb_lookup(xt, w2)
    return jnp.transpose(p, (2, 0, 1))


# speedup vs baseline: 1.5635x; 1.4572x over previous
import functools

import jax
import jax.numpy as jnp
from jax import lax
from jax.experimental import pallas as pl
from jax.experimental.pallas import tpu as pltpu
from jax.experimental.pallas import tpu_sc as plsc

BATCH = 4096
SEQ = 200
EMBED_DIM = 64
PADDED = 2 * EMBED_DIM

NUM_CORES = 2
NUM_SUBCORES = 16
NUM_WORKERS = NUM_CORES * NUM_SUBCORES

CHUNK = 128
N_STEPS = SEQ
NBUF = 2
DIST = 2
MBUF = 2
LANES = 16

_mesh = plsc.VectorSubcoreMesh(
    core_axis_name="c", subcore_axis_name="s",
    num_cores=NUM_CORES, num_subcores=NUM_SUBCORES,
)


@functools.partial(
    pl.kernel,
    out_type=jax.ShapeDtypeStruct((SEQ, EMBED_DIM, BATCH), jnp.float32),
    mesh=_mesh,
    compiler_params=pltpu.CompilerParams(needs_layout_passes=False,
                                         disable_bounds_checks=True),
    scratch_types=(
        [pltpu.VMEM((N_STEPS, CHUNK), jnp.int32),
         pltpu.VMEM((NBUF, CHUNK), jnp.int32),
         pltpu.VMEM((NBUF, CHUNK, PADDED), jnp.float32),
         pltpu.VMEM((MBUF, EMBED_DIM, CHUNK), jnp.float32)]
        + [pltpu.SemaphoreType.DMA] * NBUF
        + [pltpu.SemaphoreType.DMA] * MBUF
    ),
)
def _emb_lookup(xt_hbm, w_hbm, out_hbm, idx_v, pair_v, gbuf, tbuf, *sems):
    semg = sems[:NBUF]
    sems_ = sems[NBUF:]
    wid = lax.axis_index("s") * NUM_CORES + lax.axis_index("c")
    col0 = wid * CHUNK
    pltpu.sync_copy(xt_hbm.at[:, pl.ds(col0, CHUNK)], idx_v)

    iota = lax.iota(jnp.int32, LANES)
    c_vecs = [iota + (cg * LANES) for cg in range(CHUNK // LANES)]

    def fire_gather(s, b):
        pltpu.async_copy(w_hbm.at[idx_v.at[s]], gbuf.at[b], semg[b])

    def wait_gather(b):
        pltpu.make_async_copy(w_hbm.at[idx_v.at[0]], gbuf.at[b],
                              semg[b]).wait()

    dvs = [(iota + d) & (LANES - 1) for d in range(LANES)]

    def transpose_select(s, k, tb):
        src = gbuf.at[k]
        tdst = tbuf.at[tb]

        @plsc.parallel_loop(0, CHUNK, LANES)
        def tbody(c0):
            cv = iota + c0

            def ebody(eg, carry):
                e0 = eg * LANES
                for d in range(LANES):
                    ev = dvs[d] + e0
                    vals = plsc.load_gather(src, [cv, ev])
                    plsc.store_scatter(tdst, [ev, cv], vals)
                return carry

            lax.fori_loop(0, EMBED_DIM // LANES, ebody, 0)

    def fire_scatter(s, tb):
        pltpu.async_copy(tbuf.at[tb],
                         out_hbm.at[s, :, pl.ds(col0, CHUNK)],
                         sems_[tb])

    def wait_scatter(tb):
        pltpu.make_async_copy(tbuf.at[tb],
                              out_hbm.at[0, :, pl.ds(col0, CHUNK)],
                              sems_[tb]).wait()

    def step(s, k, warm, refill):
        tb = k % MBUF
        wait_gather(k)
        if warm:
            wait_scatter(tb)
        transpose_select(s, k, tb)
        fire_scatter(s, tb)
        if refill:
            fire_gather(s + DIST, (k + DIST) % NBUF)

    for s in range(DIST):
        fire_gather(s, s)
    for s in range(NBUF):
        step(s, s, warm=(s >= MBUF), refill=True)

    def body(g, carry):
        s0 = g * NBUF
        for k in range(NBUF):
            step(s0 + k, k, warm=True, refill=True)
        return carry

    lax.fori_loop(1, N_STEPS // NBUF - 1, body, 0)

    for k in range(NBUF):
        s = N_STEPS - NBUF + k
        step(s, k, warm=True, refill=(s + DIST < N_STEPS))
    for tb in range(MBUF):
        wait_scatter(tb)


def kernel(x, W):
    xt = jnp.transpose(x).astype(jnp.int32)
    w128 = jnp.pad(W, ((0, 0), (0, PADDED - EMBED_DIM)))
    p = _emb_lookup(xt, w128)
    return jnp.transpose(p, (2, 0, 1))


# speedup vs baseline: 1.5657x; 1.0014x over previous
import functools

import jax
import jax.numpy as jnp
from jax import lax
from jax.experimental import pallas as pl
from jax.experimental.pallas import tpu as pltpu
from jax.experimental.pallas import tpu_sc as plsc

BATCH = 4096
SEQ = 200
EMBED_DIM = 64
PADDED = 2 * EMBED_DIM

NUM_CORES = 2
NUM_SUBCORES = 16
NUM_WORKERS = NUM_CORES * NUM_SUBCORES

CHUNK = 128
N_STEPS = SEQ
NBUF = 3
DIST = 2
MBUF = 3
LANES = 16

_mesh = plsc.VectorSubcoreMesh(
    core_axis_name="c", subcore_axis_name="s",
    num_cores=NUM_CORES, num_subcores=NUM_SUBCORES,
)


@functools.partial(
    pl.kernel,
    out_type=jax.ShapeDtypeStruct((SEQ, EMBED_DIM, BATCH), jnp.float32),
    mesh=_mesh,
    compiler_params=pltpu.CompilerParams(needs_layout_passes=False,
                                         disable_bounds_checks=True),
    scratch_types=(
        [pltpu.VMEM((N_STEPS, CHUNK), jnp.int32),
         pltpu.VMEM((NBUF, CHUNK, PADDED), jnp.float32),
         pltpu.VMEM((MBUF, EMBED_DIM, CHUNK), jnp.float32)]
        + [pltpu.SemaphoreType.DMA] * NBUF
        + [pltpu.SemaphoreType.DMA] * MBUF
    ),
)
def _emb_lookup(xt_hbm, w_hbm, out_hbm, idx_v, gbuf, tbuf, *sems):
    semg = sems[:NBUF]
    sems_ = sems[NBUF:]
    wid = lax.axis_index("s") * NUM_CORES + lax.axis_index("c")
    col0 = wid * CHUNK
    pltpu.sync_copy(xt_hbm.at[:, pl.ds(col0, CHUNK)], idx_v)

    iota = lax.iota(jnp.int32, LANES)
    c_vecs = [iota + (cg * LANES) for cg in range(CHUNK // LANES)]

    def fire_gather(s, b):
        pltpu.async_copy(w_hbm.at[idx_v.at[s]], gbuf.at[b], semg[b])

    def wait_gather(b):
        pltpu.make_async_copy(w_hbm.at[idx_v.at[0]], gbuf.at[b],
                              semg[b]).wait()

    dvs = [(iota + d) & (LANES - 1) for d in range(LANES)]

    def transpose_select(s, k, tb):
        src = gbuf.at[k]
        tdst = tbuf.at[tb]

        @plsc.parallel_loop(0, CHUNK, LANES)
        def tbody(c0):
            cv = iota + c0

            def ebody(eg, carry):
                e0 = eg * LANES
                for d in range(LANES):
                    ev = dvs[d] + e0
                    vals = plsc.load_gather(src, [cv, ev])
                    plsc.store_scatter(tdst, [ev, cv], vals)
                return carry

            lax.fori_loop(0, EMBED_DIM // LANES, ebody, 0)

    def fire_scatter(s, tb):
        pltpu.async_copy(tbuf.at[tb],
                         out_hbm.at[s, :, pl.ds(col0, CHUNK)],
                         sems_[tb])

    def wait_scatter(tb):
        pltpu.make_async_copy(tbuf.at[tb],
                              out_hbm.at[0, :, pl.ds(col0, CHUNK)],
                              sems_[tb]).wait()

    def step(s, k, warm, refill):
        tb = k % MBUF
        wait_gather(k)
        if warm:
            wait_scatter(tb)
        if refill:
            fire_gather(s + DIST, (k + DIST) % NBUF)
        transpose_select(s, k, tb)
        fire_scatter(s, tb)

    for s in range(DIST):
        fire_gather(s, s)
    for s in range(DIST):
        step(s, s % NBUF, warm=False, refill=True)
    for i in range(NBUF):
        s = DIST + i
        step(s, s % NBUF, warm=(s >= MBUF), refill=True)

    first = DIST + NBUF
    groups = (N_STEPS - DIST - first) // NBUF
    def body(g, carry):
        s0 = first + g * NBUF
        for i in range(NBUF):
            s = s0 + i
            step(s, (first + i) % NBUF, warm=True, refill=True)
        return carry

    lax.fori_loop(0, groups, body, 0)

    for i in range(DIST + 1):
        s = N_STEPS - DIST - 1 + i
        step(s, s % NBUF, warm=True, refill=(s + DIST < N_STEPS))
    for tb in range(MBUF):
        wait_scatter(tb)


def kernel(x, W):
    xt = jnp.transpose(x).astype(jnp.int32)
    w128 = jnp.pad(W, ((0, 0), (0, PADDED - EMBED_DIM)))
    p = _emb_lookup(xt, w128)
    return jnp.transpose(p, (2, 0, 1))


# speedup vs baseline: 1.5890x; 1.0149x over previous
import functools

import jax
import jax.numpy as jnp
from jax import lax
from jax.experimental import pallas as pl
from jax.experimental.pallas import tpu as pltpu
from jax.experimental.pallas import tpu_sc as plsc

BATCH = 4096
SEQ = 200
EMBED_DIM = 64
PADDED = 2 * EMBED_DIM

NUM_CORES = 2
NUM_SUBCORES = 16
NUM_WORKERS = NUM_CORES * NUM_SUBCORES

CHUNK = 128
N_STEPS = SEQ
NBUF = 3
DIST = 2
MBUF = 3
LANES = 16

_mesh = plsc.VectorSubcoreMesh(
    core_axis_name="c", subcore_axis_name="s",
    num_cores=NUM_CORES, num_subcores=NUM_SUBCORES,
)


@functools.partial(
    pl.kernel,
    out_type=jax.ShapeDtypeStruct((SEQ, EMBED_DIM, BATCH), jnp.float32),
    mesh=_mesh,
    compiler_params=pltpu.CompilerParams(needs_layout_passes=False,
                                         disable_bounds_checks=True),
    scratch_types=(
        [pltpu.VMEM((N_STEPS, CHUNK), jnp.int32),
         pltpu.VMEM((NBUF, CHUNK, PADDED), jnp.float32),
         pltpu.VMEM((MBUF, EMBED_DIM, CHUNK), jnp.float32)]
        + [pltpu.SemaphoreType.DMA] * NBUF
        + [pltpu.SemaphoreType.DMA] * MBUF
    ),
)
def _emb_lookup(xt_hbm, w_hbm, out_hbm, idx_v, gbuf, tbuf, *sems):
    semg = sems[:NBUF]
    sems_ = sems[NBUF:]
    wid = lax.axis_index("s") * NUM_CORES + lax.axis_index("c")
    col0 = wid * CHUNK
    pltpu.sync_copy(xt_hbm.at[:, pl.ds(col0, CHUNK)], idx_v)

    iota = lax.iota(jnp.int32, LANES)
    c_vecs = [iota + (cg * LANES) for cg in range(CHUNK // LANES)]

    def fire_gather(s, b):
        pltpu.async_copy(w_hbm.at[idx_v.at[s]], gbuf.at[b], semg[b])

    def wait_gather(b):
        pltpu.make_async_copy(w_hbm.at[idx_v.at[0]], gbuf.at[b],
                              semg[b]).wait()

    dvs = [(iota + d) & (LANES - 1) for d in range(LANES)]

    def transpose_select(s, k, tb):
        src = gbuf.at[k]
        tdst = tbuf.at[tb]

        @plsc.parallel_loop(0, CHUNK, LANES)
        def tbody(c0):
            cv = iota + c0

            def ebody(eh, carry):
                e0 = eh * (2 * LANES)
                for sub in range(2):
                    for d in range(LANES):
                        ev = dvs[d] + (e0 + sub * LANES)
                        vals = plsc.load_gather(src, [cv, ev])
                        plsc.store_scatter(tdst, [ev, cv], vals)
                return carry

            lax.fori_loop(0, EMBED_DIM // (2 * LANES), ebody, 0)

    def fire_scatter(s, tb):
        pltpu.async_copy(tbuf.at[tb],
                         out_hbm.at[s, :, pl.ds(col0, CHUNK)],
                         sems_[tb])

    def wait_scatter(tb):
        pltpu.make_async_copy(tbuf.at[tb],
                              out_hbm.at[0, :, pl.ds(col0, CHUNK)],
                              sems_[tb]).wait()

    def step(s, k, warm, refill):
        tb = k % MBUF
        wait_gather(k)
        if warm:
            wait_scatter(tb)
        if refill:
            fire_gather(s + DIST, (k + DIST) % NBUF)
        transpose_select(s, k, tb)
        fire_scatter(s, tb)

    for s in range(DIST):
        fire_gather(s, s)
    for s in range(DIST):
        step(s, s % NBUF, warm=False, refill=True)
    for i in range(NBUF):
        s = DIST + i
        step(s, s % NBUF, warm=(s >= MBUF), refill=True)

    first = DIST + NBUF
    groups = (N_STEPS - DIST - first) // NBUF
    def body(g, carry):
        s0 = first + g * NBUF
        for i in range(NBUF):
            s = s0 + i
            step(s, (first + i) % NBUF, warm=True, refill=True)
        return carry

    lax.fori_loop(0, groups, body, 0)

    for i in range(DIST + 1):
        s = N_STEPS - DIST - 1 + i
        step(s, s % NBUF, warm=True, refill=(s + DIST < N_STEPS))
    for tb in range(MBUF):
        wait_scatter(tb)


def kernel(x, W):
    xt = jnp.transpose(x).astype(jnp.int32)
    w128 = jnp.pad(W, ((0, 0), (0, PADDED - EMBED_DIM)))
    p = _emb_lookup(xt, w128)
    return jnp.transpose(p, (2, 0, 1))
